# baseline (speedup 1.0000x reference)
import functools

import jax
import jax.numpy as jnp
from jax import lax
from jax.experimental import pallas as pl
from jax.experimental.pallas import tpu as pltpu
from jax.experimental.pallas import tpu_sc as plsc


@functools.lru_cache(maxsize=None)
def _make_gather(S, P, V, D):
    info = plsc.get_sparse_core_info()
    NC, NS = info.num_cores, info.num_subcores
    NW = NC * NS

    SPC = 4
    C = SPC * P
    G = P // 2
    K = C // G

    assert S % NW == 0
    s_per_w = S // NW
    assert s_per_w % SPC == 0
    n_chunks = s_per_w // SPC
    assert n_chunks % 2 == 0 and n_chunks >= 4
    n_qrows = S * P // C

    mesh = plsc.VectorSubcoreMesh(core_axis_name="c", subcore_axis_name="s")

    @functools.partial(
        pl.kernel,
        mesh=mesh,
        compiler_params=pltpu.CompilerParams(use_tc_tiling_on_sc=False),
        out_type=jax.ShapeDtypeStruct((S, P, D), jnp.float32),
        scratch_types=[
            pltpu.VMEM((2, 1, K, G), jnp.int32),
            pltpu.VMEM((2, SPC, P, D), jnp.float32),
            pltpu.SemaphoreType.DMA((2,)),
            pltpu.SemaphoreType.DMA,
            pltpu.SemaphoreType.DMA((2,)),
        ],
    )
    def gather_kernel(table_hbm, idx_hbm, out_hbm, idx_v, rows_v, sem_i, sem_g, sem_o):
        wid = lax.axis_index("s") * NC + lax.axis_index("c")
        q0 = wid * n_chunks
        seq0 = wid * s_per_w

        def start_idx(c, b):
            c = min(c, n_chunks - 1) if isinstance(c, int) else lax.min(c, n_chunks - 1)
            return pltpu.async_copy(
                idx_hbm.at[pl.ds(q0 + c, 1)], idx_v.at[b], sem_i.at[b]
            )

        def wait_idx(b):
            pltpu.make_async_copy(
                idx_hbm.at[pl.ds(0, 1)], idx_v.at[b], sem_i.at[b]
            ).wait()

        def run_gathers(b):
            copies = [
                pltpu.async_copy(
                    table_hbm.at[idx_v.at[b].at[0].at[j]],
                    rows_v.at[b].at[j // 2].at[pl.ds((j % 2) * G, G)],
                    sem_g,
                )
                for j in range(K)
            ]
            for cp in copies:
                cp.wait()

        def start_store(c, b):
            return pltpu.async_copy(
                rows_v.at[b], out_hbm.at[pl.ds(seq0 + c * SPC, SPC)], sem_o.at[b]
            )

        def wait_store(b):
            pltpu.make_async_copy(
                rows_v.at[b], out_hbm.at[pl.ds(seq0, SPC)], sem_o.at[b]
            ).wait()

        start_idx(0, 0)
        start_idx(1, 1)
        for b in (0, 1):
            wait_idx(b)
            run_gathers(b)
            start_store(b, b)
            start_idx(2 + b, b)

        def body(t, carry):
            for b in (0, 1):
                c = 2 * t + b
                wait_idx(b)
                wait_store(b)
                run_gathers(b)
                start_store(c, b)
                start_idx(c + 2, b)
            return carry

        lax.fori_loop(1, n_chunks // 2, body, 0)

        for b in (0, 1):
            wait_store(b)
            wait_idx(b)

    return gather_kernel, n_qrows


def kernel(positions, pe):
    S, P = positions.shape
    D = pe.shape[1]
    gather, n_qrows = _make_gather(S, P, pe.shape[0], D)
    idx = positions.reshape(n_qrows, 8, P // 2).astype(jnp.int32)
    return gather(pe.astype(jnp.float32), idx)
